# baseline (speedup 1.0000x reference)
import functools
import math

import jax
import jax.numpy as jnp
from jax import lax
from jax.experimental import pallas as pl
from jax.experimental.pallas import tpu as pltpu
from jax.experimental.pallas import tpu_sc as plsc

D_MODEL = 64
SCALE = math.sqrt(D_MODEL)

NUM_CORES = 2
NUM_SUBCORES = 16
NW = NUM_CORES * NUM_SUBCORES

GCHUNK = 128
GPB = 8
ROWS = GCHUNK * GPB


def _emb_body(x_hbm, lut_hbm, out_hbm, idx_v, rows_v, gsem):
    b_total = out_hbm.shape[0]
    per_w = b_total // NW
    n_blocks = per_w // ROWS
    wid = lax.axis_index("s") * NUM_CORES + lax.axis_index("c")
    base = wid * per_w

    def block_body(t, _):
        off = base + t * ROWS
        pltpu.sync_copy(x_hbm.at[pl.ds(off, ROWS)], idx_v)
        copies = []
        for g in range(GPB):
            copies.append(pltpu.async_copy(
                lut_hbm.at[idx_v.at[pl.ds(g * GCHUNK, GCHUNK)]],
                rows_v.at[pl.ds(g * GCHUNK, GCHUNK)],
                gsem))
        for c in copies:
            c.wait()

        def mul_body(r, _):
            for c in range(D_MODEL // 16):
                v = rows_v[r, pl.ds(c * 16, 16)]
                rows_v[r, pl.ds(c * 16, 16)] = v * SCALE
            return _

        lax.fori_loop(0, ROWS, mul_body, None, unroll=4)
        pltpu.sync_copy(rows_v, out_hbm.at[pl.ds(off, ROWS)])
        return _

    lax.fori_loop(0, n_blocks, block_body, None)


@jax.jit
def _emb_call(x_flat, lut):
    b_total = x_flat.shape[0]
    mesh = plsc.VectorSubcoreMesh(core_axis_name="c", subcore_axis_name="s")
    fn = functools.partial(
        pl.kernel,
        out_type=jax.ShapeDtypeStruct((b_total, D_MODEL), jnp.float32),
        mesh=mesh,
        scratch_types=[
            pltpu.VMEM((ROWS,), jnp.int32),
            pltpu.VMEM((ROWS, D_MODEL), jnp.float32),
            pltpu.SemaphoreType.DMA,
        ],
        compiler_params=pltpu.CompilerParams(use_tc_tiling_on_sc=False),
    )(_emb_body)
    return fn(x_flat, lut)


def kernel(x, lut):
    b, s = x.shape
    out = _emb_call(x.reshape(b * s), lut)
    return out.reshape(b, s, D_MODEL)


# speedup vs baseline: 1.0629x; 1.0629x over previous
import functools
import math

import jax
import jax.numpy as jnp
from jax import lax
from jax.experimental import pallas as pl
from jax.experimental.pallas import tpu as pltpu
from jax.experimental.pallas import tpu_sc as plsc

D_MODEL = 64
SCALE = math.sqrt(D_MODEL)

NUM_CORES = 2
NUM_SUBCORES = 16
NW = NUM_CORES * NUM_SUBCORES

CHUNK = 128
NBUF = 8


def _emb_body(x_hbm, lut_hbm, out_hbm, idx_v, *scratch):
    rows = scratch[0:NBUF]
    gsems = scratch[NBUF:2 * NBUF]
    osems = scratch[2 * NBUF:3 * NBUF]
    b_total = out_hbm.shape[0]
    per_w = b_total // NW
    n_chunks = per_w // CHUNK
    wid = lax.axis_index("s") * NUM_CORES + lax.axis_index("c")
    base = wid * per_w

    pltpu.sync_copy(x_hbm.at[pl.ds(base, per_w)], idx_v)

    def fire_gather(t, b):
        pltpu.async_copy(
            lut_hbm.at[idx_v.at[pl.ds(t * CHUNK, CHUNK)]],
            rows[b], gsems[b])

    def wait_gather(b):
        pltpu.make_async_copy(
            lut_hbm.at[idx_v.at[pl.ds(0, CHUNK)]], rows[b], gsems[b]).wait()

    def wait_scatter(b):
        pltpu.make_async_copy(
            rows[b], out_hbm.at[pl.ds(base, CHUNK)], osems[b]).wait()

    for g in range(NBUF - 1):
        fire_gather(g, g)

    def outer(tt, carry):
        for b in range(NBUF):
            t = tt * NBUF + b
            wait_gather(b)

            @plsc.parallel_loop(0, CHUNK, step=1, unroll=8)
            def _mul(r):
                for c in range(D_MODEL // 16):
                    rows[b][r, pl.ds(c * 16, 16)] = (
                        rows[b][r, pl.ds(c * 16, 16)] * SCALE)

            pltpu.async_copy(
                rows[b], out_hbm.at[pl.ds(base + t * CHUNK, CHUNK)], osems[b])

            bp = (b - 1) % NBUF

            @pl.when(t == 0)
            def _():
                fire_gather(NBUF - 1, NBUF - 1)

            @pl.when(jnp.logical_and(t >= 1, t <= n_chunks - NBUF))
            def _():
                wait_scatter(bp)
                fire_gather(t + NBUF - 1, bp)
        return carry

    lax.fori_loop(0, n_chunks // NBUF, outer, None)
    for b in range(NBUF):
        wait_scatter(b)


@jax.jit
def _emb_call(x_flat, lut):
    b_total = x_flat.shape[0]
    mesh = plsc.VectorSubcoreMesh(core_axis_name="c", subcore_axis_name="s")
    fn = functools.partial(
        pl.kernel,
        out_type=jax.ShapeDtypeStruct((b_total, D_MODEL), jnp.float32),
        mesh=mesh,
        scratch_types=[pltpu.VMEM((b_total // NW,), jnp.int32)]
        + [pltpu.VMEM((CHUNK, D_MODEL), jnp.float32) for _ in range(NBUF)]
        + [pltpu.SemaphoreType.DMA for _ in range(2 * NBUF)],
        compiler_params=pltpu.CompilerParams(use_tc_tiling_on_sc=False),
    )(_emb_body)
    return fn(x_flat, lut)


def kernel(x, lut):
    b, s = x.shape
    out = _emb_call(x.reshape(b * s), lut)
    return out.reshape(b, s, D_MODEL)


# speedup vs baseline: 1.2269x; 1.1543x over previous
import functools
import math

import jax
import jax.numpy as jnp
from jax import lax
from jax.experimental import pallas as pl
from jax.experimental.pallas import tpu as pltpu
from jax.experimental.pallas import tpu_sc as plsc

D_MODEL = 64
SCALE = math.sqrt(D_MODEL)

NUM_CORES = 2
NUM_SUBCORES = 16
NW = NUM_CORES * NUM_SUBCORES

CHUNK = 128
NBUF = 8

BV = 4096


def _tr_body(in_ref, out_ref):
    a = in_ref[...] * SCALE
    y = a.T
    z = y.reshape(BV // 2, 2, D_MODEL)
    out_ref[:, 0:D_MODEL] = z[:, 0, :]
    out_ref[:, D_MODEL:2 * D_MODEL] = z[:, 1, :]


def _sc_body(x_hbm, lut_hbm, out_hbm, idx_v, *scratch):
    rows = scratch[0:NBUF]
    gsems = scratch[NBUF:2 * NBUF]
    osems = scratch[2 * NBUF:3 * NBUF]
    b_total = out_hbm.shape[0]
    per_w = b_total // NW
    n_chunks = per_w // CHUNK
    wid = lax.axis_index("s") * NUM_CORES + lax.axis_index("c")
    base = wid * per_w

    pltpu.sync_copy(x_hbm.at[pl.ds(base, per_w)], idx_v)

    def fire_gather(t, b):
        pltpu.async_copy(
            lut_hbm.at[idx_v.at[pl.ds(t * CHUNK, CHUNK)]],
            rows[b], gsems[b])

    def wait_gather(b):
        pltpu.make_async_copy(
            lut_hbm.at[idx_v.at[pl.ds(0, CHUNK)]], rows[b], gsems[b]).wait()

    def wait_scatter(b):
        pltpu.make_async_copy(
            rows[b], out_hbm.at[pl.ds(base, CHUNK)], osems[b]).wait()

    for g in range(NBUF - 1):
        fire_gather(g, g)

    def outer(tt, carry):
        for b in range(NBUF):
            t = tt * NBUF + b
            wait_gather(b)
            pltpu.async_copy(
                rows[b], out_hbm.at[pl.ds(base + t * CHUNK, CHUNK)], osems[b])

            bp = (b - 1) % NBUF

            @pl.when(t == 0)
            def _():
                fire_gather(NBUF - 1, NBUF - 1)

            @pl.when(jnp.logical_and(t >= 1, t <= n_chunks - NBUF))
            def _():
                wait_scatter(bp)
                fire_gather(t + NBUF - 1, bp)
        return carry

    lax.fori_loop(0, n_chunks // NBUF, outer, None)
    for b in range(NBUF):
        wait_scatter(b)


@jax.jit
def _emb_call(x, lut):
    b, s = x.shape
    b_total = b * s
    v, d = lut.shape

    lut_t = lut.T
    grid = (v + BV - 1) // BV
    scaled2 = pl.pallas_call(
        _tr_body,
        grid=(grid,),
        in_specs=[pl.BlockSpec((d, BV), lambda j: (0, j))],
        out_specs=pl.BlockSpec((BV // 2, 2 * d), lambda j: (j, 0)),
        out_shape=jax.ShapeDtypeStruct((v // 2, 2 * d), jnp.float32),
    )(lut_t)
    table = scaled2.reshape(v, d)

    x_flat = x.reshape(b_total)
    mesh = plsc.VectorSubcoreMesh(core_axis_name="c", subcore_axis_name="s")
    gather_fn = functools.partial(
        pl.kernel,
        out_type=jax.ShapeDtypeStruct((b_total, d), jnp.float32),
        mesh=mesh,
        scratch_types=[pltpu.VMEM((b_total // NW,), jnp.int32)]
        + [pltpu.VMEM((CHUNK, d), jnp.float32) for _ in range(NBUF)]
        + [pltpu.SemaphoreType.DMA for _ in range(2 * NBUF)],
        compiler_params=pltpu.CompilerParams(use_tc_tiling_on_sc=False),
    )(_sc_body)
    out = gather_fn(x_flat, table)
    return out.reshape(b, s, d)


def kernel(x, lut):
    return _emb_call(x, lut)
